# baseline (speedup 1.0000x reference)
import functools

import jax
import jax.numpy as jnp
from jax import lax
from jax.experimental import pallas as pl
from jax.experimental.pallas import tpu as pltpu
from jax.experimental.pallas import tpu_sc as plsc

N = 100000
E = 1600000
NC = 2
NS = 16
B = 80

TILE_ROWS = 6240
LAST_ROWS = N - 15 * TILE_ROWS
ZC = 160

_mesh = plsc.VectorSubcoreMesh(
    core_axis_name="c", subcore_axis_name="s", num_cores=NC, num_subcores=NS)


def _tile_rows(t):
    base = t * TILE_ROWS
    trips = jnp.where(t == 15, LAST_ROWS // ZC, TILE_ROWS // ZC)
    return base, trips


def _zero_chunk(zbuf, rows):
    def zb(i, _):
        zbuf[pl.ds(i * 16, 16)] = jnp.zeros((16,), jnp.float32)
        return 0
    lax.fori_loop(0, rows // 16, zb, 0)


def _zero_chunk2(zbuf, rows):
    def zb(i, _):
        zbuf[i, :] = jnp.zeros((16,), jnp.float32)
        return 0
    lax.fori_loop(0, rows, zb, 0)


CH = 128
KJ = 5
ER = E // CH
G16A, G16B = 156, 160
G1A, G1B = 78, 80


def _edge_groups(t, ga, gb):
    return t * ga * KJ, jnp.where(t == 15, gb, ga)


def _sc_degree_body(row_hbm, out_hbm, acc, rbuf, ones_v, zbuf, isem, ssem):
    c = lax.axis_index("c")
    t = lax.axis_index("s")
    base, trips = _tile_rows(t)

    _zero_chunk(zbuf, ZC)
    def onesb(i, _):
        ones_v[pl.ds(i * 16, 16)] = jnp.ones((16,), jnp.float32)
        return 0
    lax.fori_loop(0, CH // 16, onesb, 0)

    def zero_acc(i, _):
        pltpu.sync_copy(zbuf, acc.at[pl.ds(base + i * ZC, ZC)])
        return 0
    lax.fori_loop(0, trips, zero_acc, 0)
    plsc.subcore_barrier()

    row0, egroups = _edge_groups(t, G1A, G1B)
    row0 = row0 + c * (ER // NC)

    def idx_cp(g):
        return pltpu.make_async_copy(
            row_hbm.at[pl.ds(row0 + g * KJ, KJ), :], rbuf.at[g % 4],
            isem.at[g % 4])

    idx_cp(0).start()
    idx_cp(1).start()

    def grp(g, _):
        idx_cp(g).wait()
        for j in range(KJ):
            @pl.when(g >= 2)
            def _():
                pltpu.make_async_copy(
                    ones_v, acc.at[rbuf.at[g % 4, j]], ssem.at[g % 2, j]
                ).wait()
        @pl.when(g + 2 < egroups)
        def _():
            idx_cp(g + 2).start()
        for j in range(KJ):
            pltpu.async_copy(ones_v, acc.at[rbuf.at[g % 4, j]],
                             ssem.at[g % 2, j], add=True)
        return 0
    lax.fori_loop(0, egroups, grp, 0)
    for q in range(2):
        for j in range(KJ):
            pltpu.make_async_copy(
                ones_v, acc.at[rbuf.at[q, j]], ssem.at[q, j]).wait()
    plsc.subcore_barrier()

    def wout(i, _):
        o = base + i * ZC
        pltpu.sync_copy(acc.at[pl.ds(o, ZC)], zbuf)
        pltpu.sync_copy(zbuf, out_hbm.at[pl.ds(c * N + o, ZC)])
        return 0
    lax.fori_loop(0, trips, wout, 0)


_sc_degree = pl.kernel(
    _sc_degree_body,
    out_type=jax.ShapeDtypeStruct((NC * N,), jnp.float32),
    mesh=_mesh,
    compiler_params=pltpu.CompilerParams(use_tc_tiling_on_sc=False),
    scratch_types=[
        pltpu.VMEM_SHARED((N,), jnp.float32),
        pltpu.VMEM((4, KJ, CH), jnp.int32),
        pltpu.VMEM((CH,), jnp.float32),
        pltpu.VMEM((ZC,), jnp.float32),
        pltpu.SemaphoreType.DMA((4,)),
        pltpu.SemaphoreType.DMA((2, KJ)),
    ],
)


def _prop_pipeline(row_hbm, col_hbm, z_hbm, acc, rbuf, cbuf, gbuf,
                   isem, gsem, ssem, rrow0, crow0, egroups):
    def idx_cp(g):
        return [pltpu.make_async_copy(
                    row_hbm.at[pl.ds(rrow0 + g * KJ, KJ), :],
                    rbuf.at[g % 4], isem.at[g % 4]),
                pltpu.make_async_copy(
                    col_hbm.at[pl.ds(crow0 + g * KJ, KJ), :],
                    cbuf.at[g % 4], isem.at[g % 4])]

    def gath_cp(g, j):
        return pltpu.make_async_copy(
            z_hbm.at[rbuf.at[g % 4, j]], gbuf.at[g % 2, j],
            gsem.at[g % 2, j])

    def scat_cp(g, j):
        return pltpu.make_async_copy(
            gbuf.at[g % 2, j], acc.at[cbuf.at[g % 4, j]], ssem.at[g % 2, j])

    for d in idx_cp(0):
        d.start()
    for d in idx_cp(1):
        d.start()
    for d in idx_cp(0):
        d.wait()
    for j in range(KJ):
        gath_cp(0, j).start()

    def grp(g, _):
        @pl.when(g + 1 < egroups)
        def _():
            for d in idx_cp(g + 1):
                d.wait()
        @pl.when(g + 2 < egroups)
        def _():
            for d in idx_cp(g + 2):
                d.start()
        for j in range(KJ):
            @pl.when(g >= 1)
            def _():
                scat_cp(g - 1, j).wait()
            @pl.when(g + 1 < egroups)
            def _():
                gath_cp(g + 1, j).start()
        for j in range(KJ):
            gath_cp(g, j).wait()
            pltpu.async_copy(gbuf.at[g % 2, j], acc.at[cbuf.at[g % 4, j]],
                             ssem.at[g % 2, j], add=True)
        return 0
    lax.fori_loop(0, egroups, grp, 0)
    for j in range(KJ):
        scat_cp(egroups - 1, j).wait()


def _sc_prop1_body(row_hbm, col_hbm, z_hbm, out_hbm, acc, rbuf, cbuf, gbuf,
                   zbuf, isem, gsem, ssem):
    c = lax.axis_index("c")
    t = lax.axis_index("s")
    base, trips = _tile_rows(t)

    _zero_chunk(zbuf, ZC)
    def zero_acc(i, _):
        pltpu.sync_copy(zbuf, acc.at[pl.ds(base + i * ZC, ZC)])
        return 0
    lax.fori_loop(0, trips, zero_acc, 0)
    plsc.subcore_barrier()

    row0, egroups = _edge_groups(t, G1A, G1B)
    row0 = row0 + c * (ER // NC)
    _prop_pipeline(row_hbm, col_hbm, z_hbm, acc, rbuf, cbuf, gbuf,
                   isem, gsem, ssem, row0, row0, egroups)
    plsc.subcore_barrier()

    def wout(i, _):
        o = base + i * ZC
        pltpu.sync_copy(acc.at[pl.ds(o, ZC)], zbuf)
        pltpu.sync_copy(zbuf, out_hbm.at[pl.ds(c * N + o, ZC)])
        return 0
    lax.fori_loop(0, trips, wout, 0)


_sc_prop1 = pl.kernel(
    _sc_prop1_body,
    out_type=jax.ShapeDtypeStruct((NC * N,), jnp.float32),
    mesh=_mesh,
    compiler_params=pltpu.CompilerParams(use_tc_tiling_on_sc=False),
    scratch_types=[
        pltpu.VMEM_SHARED((N,), jnp.float32),
        pltpu.VMEM((4, KJ, CH), jnp.int32),
        pltpu.VMEM((4, KJ, CH), jnp.int32),
        pltpu.VMEM((2, KJ, CH), jnp.float32),
        pltpu.VMEM((ZC,), jnp.float32),
        pltpu.SemaphoreType.DMA((4,)),
        pltpu.SemaphoreType.DMA((2, KJ)),
        pltpu.SemaphoreType.DMA((2, KJ)),
    ],
)


def _sc_prop16_body(rowb_hbm, col_hbm, z_hbm, out_hbm, acc, rbuf, cbuf, gbuf,
                    zbuf, isem, gsem, ssem):
    c = lax.axis_index("c")
    t = lax.axis_index("s")
    base, trips = _tile_rows(t)

    _zero_chunk2(zbuf, ZC)
    def zero_acc(i, _):
        pltpu.sync_copy(zbuf, acc.at[pl.ds(base + i * ZC, ZC), :])
        return 0
    lax.fori_loop(0, trips, zero_acc, 0)
    plsc.subcore_barrier()

    crow0, egroups = _edge_groups(t, G16A, G16B)
    rrow0 = crow0 + c * ER
    _prop_pipeline(rowb_hbm, col_hbm, z_hbm, acc, rbuf, cbuf, gbuf,
                   isem, gsem, ssem, rrow0, crow0, egroups)
    plsc.subcore_barrier()

    def wout(i, _):
        o = base + i * ZC
        pltpu.sync_copy(acc.at[pl.ds(o, ZC), :], zbuf)
        pltpu.sync_copy(zbuf, out_hbm.at[pl.ds(c * N + o, ZC), :])
        return 0
    lax.fori_loop(0, trips, wout, 0)


def _sc_prop16s_body(rowb_hbm, col_hbm, z_hbm, invd_hbm, out_hbm, zs2_hbm,
                     acc, rbuf, cbuf, gbuf, zbuf, ibuf, isem, gsem, ssem):
    c = lax.axis_index("c")
    t = lax.axis_index("s")
    base, trips = _tile_rows(t)

    _zero_chunk2(zbuf, ZC)
    def zero_acc(i, _):
        pltpu.sync_copy(zbuf, acc.at[pl.ds(base + i * ZC, ZC), :])
        return 0
    lax.fori_loop(0, trips, zero_acc, 0)
    plsc.subcore_barrier()

    crow0, egroups = _edge_groups(t, G16A, G16B)
    rrow0 = crow0 + c * ER
    _prop_pipeline(rowb_hbm, col_hbm, z_hbm, acc, rbuf, cbuf, gbuf,
                   isem, gsem, ssem, rrow0, crow0, egroups)
    plsc.subcore_barrier()

    def wout(i, _):
        o = base + i * ZC
        pltpu.sync_copy(acc.at[pl.ds(o, ZC), :], zbuf)
        pltpu.sync_copy(zbuf, out_hbm.at[pl.ds(c * N + o, ZC), :])
        pltpu.sync_copy(invd_hbm.at[pl.ds(o, ZC)], ibuf)
        def scale(q, _):
            iv = ibuf[pl.ds(q * 16, 16)]
            for k in range(16):
                zbuf[q * 16 + k, :] = zbuf[q * 16 + k, :] * (-iv[k])
            return 0
        lax.fori_loop(0, ZC // 16, scale, 0)
        pltpu.sync_copy(zbuf, zs2_hbm.at[pl.ds(c * N + o, ZC), :])
        return 0
    lax.fori_loop(0, trips, wout, 0)


_sc_prop16s = pl.kernel(
    _sc_prop16s_body,
    out_type=[jax.ShapeDtypeStruct((NC * N, 16), jnp.float32),
              jax.ShapeDtypeStruct((NC * N, 16), jnp.float32)],
    mesh=_mesh,
    compiler_params=pltpu.CompilerParams(use_tc_tiling_on_sc=False),
    scratch_types=[
        pltpu.VMEM_SHARED((N, 16), jnp.float32),
        pltpu.VMEM((4, KJ, CH), jnp.int32),
        pltpu.VMEM((4, KJ, CH), jnp.int32),
        pltpu.VMEM((2, KJ, CH, 16), jnp.float32),
        pltpu.VMEM((ZC, 16), jnp.float32),
        pltpu.VMEM((ZC,), jnp.float32),
        pltpu.SemaphoreType.DMA((4,)),
        pltpu.SemaphoreType.DMA((2, KJ)),
        pltpu.SemaphoreType.DMA((2, KJ)),
    ],
)


_sc_prop16 = pl.kernel(
    _sc_prop16_body,
    out_type=jax.ShapeDtypeStruct((NC * N, 16), jnp.float32),
    mesh=_mesh,
    compiler_params=pltpu.CompilerParams(use_tc_tiling_on_sc=False),
    scratch_types=[
        pltpu.VMEM_SHARED((N, 16), jnp.float32),
        pltpu.VMEM((4, KJ, CH), jnp.int32),
        pltpu.VMEM((4, KJ, CH), jnp.int32),
        pltpu.VMEM((2, KJ, CH, 16), jnp.float32),
        pltpu.VMEM((ZC, 16), jnp.float32),
        pltpu.SemaphoreType.DMA((4,)),
        pltpu.SemaphoreType.DMA((2, KJ)),
        pltpu.SemaphoreType.DMA((2, KJ)),
    ],
)


BN = 5000


def _tc_norm_body(degp_ref, x0_ref, z1_ref, dis_ref, disp_ref, invd_ref,
                  invp_ref):
    deg = degp_ref[pl.ds(0, N)] + degp_ref[pl.ds(N, N)]
    pos = deg > 0
    dis = jnp.where(pos, lax.rsqrt(jnp.maximum(deg, 1.0)), 0.0)
    dis_ref[...] = dis
    disp_ref[...] = jnp.where(pos, dis, 1.0)
    invd_ref[...] = dis * dis
    invp_ref[...] = jnp.where(pos, jnp.sqrt(jnp.maximum(deg, 1.0)), 1.0)
    z1_ref[...] = dis * x0_ref[...]


def _tc_norm(degp, x0):
    return pl.pallas_call(
        _tc_norm_body,
        out_shape=[jax.ShapeDtypeStruct((N,), jnp.float32)] * 5,
    )(degp, x0)


def _tc_scale1_body(a1p_ref, dis_ref, invd_ref, t1_ref, z2_ref):
    a1 = a1p_ref[pl.ds(0, N)] + a1p_ref[pl.ds(N, N)]
    t1_ref[...] = -dis_ref[...] * a1
    z2_ref[...] = -invd_ref[...] * a1


def _tc_scale1(a1p, dis, invd):
    return pl.pallas_call(
        _tc_scale1_body,
        out_shape=[jax.ShapeDtypeStruct((N,), jnp.float32)] * 2,
    )(a1p, dis, invd)


def _tc_pre_body(a2p_ref, dis_ref, x0_ref, t1_ref, disp_ref,
                 c0_ref, c1_ref, c2_ref):
    a2 = a2p_ref[pl.ds(0, N)] + a2p_ref[pl.ds(N, N)]
    x0 = x0_ref[...]
    disp = disp_ref[...]
    t2 = -2.0 * dis_ref[...] * a2 - x0
    c0_ref[...] = disp * x0
    c1_ref[...] = disp * t1_ref[...]
    c2_ref[...] = disp * t2


def _tc_pre(a2p, dis, x0, t1, disp):
    return pl.pallas_call(
        _tc_pre_body,
        out_shape=[jax.ShapeDtypeStruct((N,), jnp.float32)] * 3,
    )(a2p, dis, x0, t1, disp)


def _tc_conv1_body(x4_ref, w4_ref, zs_ref):
    zs_ref[...] = jnp.maximum(
        jnp.dot(x4_ref[...], w4_ref[0], preferred_element_type=jnp.float32),
        0.0)


def _tc_conv1(x4, w4):
    return pl.pallas_call(
        _tc_conv1_body,
        grid=(2, N // BN),
        in_specs=[
            pl.BlockSpec((BN, 4), lambda c, i: (i, 0)),
            pl.BlockSpec((1, 4, 16), lambda c, i: (c, 0, 0)),
        ],
        out_specs=pl.BlockSpec((BN, 16), lambda c, i: (c * (N // BN) + i, 0)),
        out_shape=jax.ShapeDtypeStruct((2 * N, 16), jnp.float32),
    )(x4, w4)


def _tc_zs2_body(av1_ref, invd_ref, zs2_ref):
    zs2_ref[...] = -invd_ref[...] * av1_ref[...]


def _tc_zs2(av1, invd1):
    return pl.pallas_call(
        _tc_zs2_body,
        grid=(2, N // BN),
        in_specs=[
            pl.BlockSpec((BN, 16), lambda c, i: (c * (N // BN) + i, 0)),
            pl.BlockSpec((BN, 1), lambda c, i: (i, 0)),
        ],
        out_specs=pl.BlockSpec((BN, 16), lambda c, i: (c * (N // BN) + i, 0)),
        out_shape=jax.ShapeDtypeStruct((2 * N, 16), jnp.float32),
    )(av1, invd1)


def _tc_head_body(zlo_ref, zhi_ref, a1lo_ref, a1hi_ref, a2lo_ref, a2hi_ref,
                  invp_ref, dis_ref, m0_ref, w12_ref, b2_ref,
                  wfc_ref, bfc_ref, out_ref):
    dot = lambda a, b: jnp.dot(a, b, preferred_element_type=jnp.float32)
    zc = jnp.concatenate([zlo_ref[...], zhi_ref[...]], axis=1)
    ac = jnp.concatenate([a1lo_ref[...], a1hi_ref[...],
                          2.0 * a2lo_ref[...], 2.0 * a2hi_ref[...]], axis=1)
    p0 = dot(zc, m0_ref[...])
    p1 = dot(ac, w12_ref[...])
    g = jnp.maximum(invp_ref[...] * p0 - dis_ref[...] * p1
                    + b2_ref[...][None, :], 0.0)
    out_ref[...] = (jnp.sum(g * wfc_ref[0, :][None, :], axis=1,
                            keepdims=True) + bfc_ref[...][None, :])


def _tc_head(zs1, av1, av2, invp1, dis1, m0, w12, b2, wfc, bfc):
    half = pl.BlockSpec((BN, 16), lambda i: (i, 0))
    hihalf = pl.BlockSpec((BN, 16), lambda i: (N // BN + i, 0))
    col = pl.BlockSpec((BN, 1), lambda i: (i, 0))
    full = lambda shp: pl.BlockSpec(shp, lambda i: tuple(0 for _ in shp))
    return pl.pallas_call(
        _tc_head_body,
        grid=(N // BN,),
        in_specs=[half, hihalf, half, hihalf, half, hihalf, col, col,
                  full((32, 32)), full((64, 32)),
                  full((32,)), full((1, 32)), full((1,))],
        out_specs=pl.BlockSpec((BN, 1), lambda i: (i, 0)),
        out_shape=jax.ShapeDtypeStruct((N, 1), jnp.float32),
    )(zs1, zs1, av1, av1, av2, av2, invp1, dis1, m0, w12, b2, wfc, bfc)


def kernel(x, edge_index, W1, b1, W2, b2, Wfc, bfc):
    row = edge_index[0]
    col = edge_index[1]
    row2 = row.reshape(ER, CH)
    col2 = col.reshape(ER, CH)
    rowb2 = jnp.concatenate([row, row + N]).reshape(2 * ER, CH)
    x0 = x[:, 0]

    degp = _sc_degree(row2)
    z1, dis, disp, invd, invp = _tc_norm(degp, x0)

    a1p = _sc_prop1(row2, col2, z1)
    t1, z2 = _tc_scale1(a1p, dis, invd)

    a2p = _sc_prop1(row2, col2, z2)
    c0, c1, c2 = _tc_pre(a2p, dis, x0, t1, disp)
    x4 = jnp.stack([c0, c1, c2, disp], axis=1)
    w4 = jnp.concatenate([W1.reshape(3, 32), b1[None, :]], axis=0)
    w4s = jnp.stack([w4[:, :16], w4[:, 16:]])
    zs1 = _tc_conv1(x4, w4s)

    av1, zs2 = _sc_prop16s(rowb2, col2, zs1, invd)

    av2 = _sc_prop16(rowb2, col2, zs2)
    return _tc_head(zs1, av1, av2, invp.reshape(N, 1), dis.reshape(N, 1),
                    W2[0] - W2[2], jnp.concatenate([W2[1], W2[2]], axis=0),
                    b2, Wfc.reshape(1, 32), bfc)
